# Initial kernel scaffold; baseline (speedup 1.0000x reference)
import functools

import jax
import jax.numpy as jnp
from jax import lax
from jax.experimental import pallas as pl
from jax.experimental.pallas import tpu as pltpu

F = 2


def _topk2_body(focus_ref, inds_ref, vals_ref):
    f = focus_ref[...]
    bsz, secl = f.shape
    iota = lax.broadcasted_iota(jnp.int32, (bsz, secl), 1)
    m1 = jnp.max(f, axis=1, keepdims=True)
    i1 = jnp.min(jnp.where(f == m1, iota, secl), axis=1, keepdims=True)
    f2 = jnp.where(iota == i1, -jnp.inf, f)
    m2 = jnp.max(f2, axis=1, keepdims=True)
    i2 = jnp.min(jnp.where(f2 == m2, iota, secl), axis=1, keepdims=True)
    inds_ref[...] = jnp.concatenate([i1, i2], axis=1)
    vals_ref[...] = jnp.concatenate([m1, m2], axis=1)


def _attn_body(inds_ref, vals_ref,
               dec_h_ref, wd_ref, bdec_ref, wv_ref, wcov_ref,
               ef0_ref, ef1_ref, eo0_ref, eo1_ref,
               c0_ref, c1_ref, cov_ref, mask_ref,
               ctx_ref, attn_ref, covout_ref):
    b = pl.program_id(0)
    i0 = inds_ref[b, 0]
    i1 = inds_ref[b, 1]
    v0 = vals_ref[b, 0]
    v1 = vals_ref[b, 1]

    dec = lax.dot_general(
        dec_h_ref[...], wd_ref[...], (((1,), (1,)), ((), ())),
        preferred_element_type=jnp.float32,
        precision=lax.Precision.HIGHEST)
    dec = dec + bdec_ref[...]

    wcov = wcov_ref[...]
    x0 = ef0_ref[0, 0] + dec + c0_ref[0, 0] * wcov
    x1 = ef1_ref[0, 0] + dec + c1_ref[0, 0] * wcov
    t = jnp.concatenate([jnp.tanh(x0), jnp.tanh(x1)], axis=0)

    s = lax.dot_general(
        wv_ref[...], t, (((1,), (1,)), ((), ())),
        preferred_element_type=jnp.float32,
        precision=lax.Precision.HIGHEST)

    wordl = x0.shape[0]
    m0 = mask_ref[0, pl.ds(i0, 1), :]
    m1 = mask_ref[0, pl.ds(i1, 1), :]
    mask_row = jnp.concatenate([m0, m1], axis=1)
    foc_row = jnp.concatenate(
        [jnp.full((1, wordl), v0, jnp.float32),
         jnp.full((1, wordl), v1, jnp.float32)], axis=1)

    e = jnp.exp(s - jnp.max(s))
    af = e * mask_row * foc_row
    w = af / jnp.sum(af)

    ctx_ref[...] = lax.dot_general(
        w[:, :wordl], eo0_ref[0, 0], (((1,), (0,)), ((), ())),
        preferred_element_type=jnp.float32,
        precision=lax.Precision.HIGHEST) + lax.dot_general(
        w[:, wordl:], eo1_ref[0, 0], (((1,), (0,)), ((), ())),
        preferred_element_type=jnp.float32,
        precision=lax.Precision.HIGHEST)

    attn_ref[0] = jnp.zeros_like(attn_ref[0])
    attn_ref[0, pl.ds(i0, 1), :] = w[:, :wordl]
    attn_ref[0, pl.ds(i1, 1), :] = w[:, wordl:]
    covout_ref[0] = cov_ref[0] + attn_ref[0]


def kernel(dec_hidden, enc_output, enc_feature, enc_mask, sec_attn, coverage,
           focus, W_dec, b_dec, w_v, w_cov):
    batch, src_len, dim = enc_output.shape
    secl = focus.shape[1]
    wordl = src_len // secl

    inds, vals = pl.pallas_call(
        _topk2_body,
        out_shape=(jax.ShapeDtypeStruct((batch, F), jnp.int32),
                   jax.ShapeDtypeStruct((batch, F), jnp.float32)),
    )(focus)

    ef = enc_feature.reshape(batch, secl, wordl, dim)
    eo = enc_output.reshape(batch, secl, wordl, dim)
    cov4 = coverage.reshape(batch, secl, wordl, 1)
    cov3 = coverage.reshape(batch, secl, wordl)
    mask3 = enc_mask.reshape(batch, secl, wordl)

    sec_spec = lambda f_slot: pl.BlockSpec(
        (1, 1, wordl, dim), lambda b, ir, vr, _f=f_slot: (b, ir[b, _f], 0, 0))
    cov_sec_spec = lambda f_slot: pl.BlockSpec(
        (1, 1, wordl, 1), lambda b, ir, vr, _f=f_slot: (b, ir[b, _f], 0, 0))

    grid_spec = pltpu.PrefetchScalarGridSpec(
        num_scalar_prefetch=2,
        grid=(batch,),
        in_specs=[
            pl.BlockSpec((1, dim), lambda b, ir, vr: (b, 0)),
            pl.BlockSpec((dim, dim), lambda b, ir, vr: (0, 0)),
            pl.BlockSpec((1, dim), lambda b, ir, vr: (0, 0)),
            pl.BlockSpec((1, dim), lambda b, ir, vr: (0, 0)),
            pl.BlockSpec((1, dim), lambda b, ir, vr: (0, 0)),
            sec_spec(0), sec_spec(1),
            sec_spec(0), sec_spec(1),
            cov_sec_spec(0), cov_sec_spec(1),
            pl.BlockSpec((1, secl, wordl), lambda b, ir, vr: (b, 0, 0)),
            pl.BlockSpec((1, secl, wordl), lambda b, ir, vr: (b, 0, 0)),
        ],
        out_specs=[
            pl.BlockSpec((1, dim), lambda b, ir, vr: (b, 0)),
            pl.BlockSpec((1, secl, wordl), lambda b, ir, vr: (b, 0, 0)),
            pl.BlockSpec((1, secl, wordl), lambda b, ir, vr: (b, 0, 0)),
        ],
    )

    context, attn3, covout3 = pl.pallas_call(
        _attn_body,
        grid_spec=grid_spec,
        out_shape=(jax.ShapeDtypeStruct((batch, dim), jnp.float32),
                   jax.ShapeDtypeStruct((batch, secl, wordl), jnp.float32),
                   jax.ShapeDtypeStruct((batch, secl, wordl), jnp.float32)),
        compiler_params=pltpu.CompilerParams(
            dimension_semantics=("arbitrary",)),
    )(inds, vals, dec_hidden, W_dec, b_dec.reshape(1, dim),
      w_v.reshape(1, dim), w_cov.reshape(1, dim),
      ef, ef, eo, eo, cov4, cov4, cov3, mask3)

    return (context, attn3.reshape(batch, src_len),
            covout3.reshape(batch, src_len))


# speedup vs baseline: 1.0021x; 1.0021x over previous
import functools

import jax
import jax.numpy as jnp
from jax import lax
from jax.experimental import pallas as pl
from jax.experimental.pallas import tpu as pltpu

F = 2


def _topk2_body(focus_ref, inds_ref, vals_ref):
    f = focus_ref[...]
    bsz, secl = f.shape
    iota = lax.broadcasted_iota(jnp.int32, (bsz, secl), 1)
    m1 = jnp.max(f, axis=1, keepdims=True)
    i1 = jnp.min(jnp.where(f == m1, iota, secl), axis=1, keepdims=True)
    f2 = jnp.where(iota == i1, -jnp.inf, f)
    m2 = jnp.max(f2, axis=1, keepdims=True)
    i2 = jnp.min(jnp.where(f2 == m2, iota, secl), axis=1, keepdims=True)
    inds_ref[...] = jnp.concatenate([i1, i2], axis=1)
    vals_ref[...] = jnp.concatenate([m1, m2], axis=1)


def _attn_body(inds_ref, vals_ref,
               dec_h_ref, wd_ref, bdec_ref, wv_ref, wcov_ref,
               ef0_ref, ef1_ref, eo0_ref, eo1_ref,
               c0_ref, c1_ref, cov_ref, mask_ref,
               ctx_ref, attn_ref, covout_ref):
    b = pl.program_id(0)
    i0 = inds_ref[b, 0]
    i1 = inds_ref[b, 1]
    v0 = vals_ref[b, 0]
    v1 = vals_ref[b, 1]

    dec = lax.dot_general(
        dec_h_ref[0], wd_ref[...], (((1,), (1,)), ((), ())),
        preferred_element_type=jnp.float32,
        precision=lax.Precision.HIGHEST)
    dec = dec + bdec_ref[...]

    wcov = wcov_ref[...]
    x0 = ef0_ref[0, 0] + dec + c0_ref[0, 0] * wcov
    x1 = ef1_ref[0, 0] + dec + c1_ref[0, 0] * wcov
    t = jnp.concatenate([jnp.tanh(x0), jnp.tanh(x1)], axis=0)

    s = lax.dot_general(
        wv_ref[...], t, (((1,), (1,)), ((), ())),
        preferred_element_type=jnp.float32,
        precision=lax.Precision.HIGHEST)

    wordl = x0.shape[0]
    m0 = mask_ref[0, pl.ds(i0, 1), :]
    m1 = mask_ref[0, pl.ds(i1, 1), :]
    mask_row = jnp.concatenate([m0, m1], axis=1)
    foc_row = jnp.concatenate(
        [jnp.full((1, wordl), v0, jnp.float32),
         jnp.full((1, wordl), v1, jnp.float32)], axis=1)

    e = jnp.exp(s - jnp.max(s))
    af = e * mask_row * foc_row
    w = af / jnp.sum(af)

    ctx_ref[0] = lax.dot_general(
        w[:, :wordl], eo0_ref[0, 0], (((1,), (0,)), ((), ())),
        preferred_element_type=jnp.float32,
        precision=lax.Precision.HIGHEST) + lax.dot_general(
        w[:, wordl:], eo1_ref[0, 0], (((1,), (0,)), ((), ())),
        preferred_element_type=jnp.float32,
        precision=lax.Precision.HIGHEST)

    attn_ref[0] = jnp.zeros_like(attn_ref[0])
    attn_ref[0, pl.ds(i0, 1), :] = w[:, :wordl]
    attn_ref[0, pl.ds(i1, 1), :] = w[:, wordl:]
    covout_ref[0] = cov_ref[0] + attn_ref[0]


def kernel(dec_hidden, enc_output, enc_feature, enc_mask, sec_attn, coverage,
           focus, W_dec, b_dec, w_v, w_cov):
    batch, src_len, dim = enc_output.shape
    secl = focus.shape[1]
    wordl = src_len // secl

    inds, vals = pl.pallas_call(
        _topk2_body,
        out_shape=(jax.ShapeDtypeStruct((batch, F), jnp.int32),
                   jax.ShapeDtypeStruct((batch, F), jnp.float32)),
    )(focus)

    ef = enc_feature.reshape(batch, secl, wordl, dim)
    eo = enc_output.reshape(batch, secl, wordl, dim)
    cov4 = coverage.reshape(batch, secl, wordl, 1)
    cov3 = coverage.reshape(batch, secl, wordl)
    mask3 = enc_mask.reshape(batch, secl, wordl)

    sec_spec = lambda f_slot: pl.BlockSpec(
        (1, 1, wordl, dim), lambda b, ir, vr, _f=f_slot: (b, ir[b, _f], 0, 0))
    cov_sec_spec = lambda f_slot: pl.BlockSpec(
        (1, 1, wordl, 1), lambda b, ir, vr, _f=f_slot: (b, ir[b, _f], 0, 0))

    grid_spec = pltpu.PrefetchScalarGridSpec(
        num_scalar_prefetch=2,
        grid=(batch,),
        in_specs=[
            pl.BlockSpec((1, 1, dim), lambda b, ir, vr: (b, 0, 0)),
            pl.BlockSpec((dim, dim), lambda b, ir, vr: (0, 0)),
            pl.BlockSpec((1, dim), lambda b, ir, vr: (0, 0)),
            pl.BlockSpec((1, dim), lambda b, ir, vr: (0, 0)),
            pl.BlockSpec((1, dim), lambda b, ir, vr: (0, 0)),
            sec_spec(0), sec_spec(1),
            sec_spec(0), sec_spec(1),
            cov_sec_spec(0), cov_sec_spec(1),
            pl.BlockSpec((1, secl, wordl), lambda b, ir, vr: (b, 0, 0)),
            pl.BlockSpec((1, secl, wordl), lambda b, ir, vr: (b, 0, 0)),
        ],
        out_specs=[
            pl.BlockSpec((1, 1, dim), lambda b, ir, vr: (b, 0, 0)),
            pl.BlockSpec((1, secl, wordl), lambda b, ir, vr: (b, 0, 0)),
            pl.BlockSpec((1, secl, wordl), lambda b, ir, vr: (b, 0, 0)),
        ],
    )

    context, attn3, covout3 = pl.pallas_call(
        _attn_body,
        grid_spec=grid_spec,
        out_shape=(jax.ShapeDtypeStruct((batch, 1, dim), jnp.float32),
                   jax.ShapeDtypeStruct((batch, secl, wordl), jnp.float32),
                   jax.ShapeDtypeStruct((batch, secl, wordl), jnp.float32)),
        compiler_params=pltpu.CompilerParams(
            dimension_semantics=("arbitrary",)),
    )(inds, vals, dec_hidden.reshape(batch, 1, dim), W_dec,
      b_dec.reshape(1, dim), w_v.reshape(1, dim), w_cov.reshape(1, dim),
      ef, ef, eo, eo, cov4, cov4, cov3, mask3)

    return (context.reshape(batch, dim), attn3.reshape(batch, src_len),
            covout3.reshape(batch, src_len))


# speedup vs baseline: 5.7771x; 5.7651x over previous
import jax
import jax.numpy as jnp
from jax import lax
from jax.experimental import pallas as pl
from jax.experimental.pallas import tpu as pltpu

F = 2


def _topk2_body(focus_ref, inds_ref, vals_ref):
    f = focus_ref[...]
    bsz, secl = f.shape
    iota = lax.broadcasted_iota(jnp.int32, (bsz, secl), 1)
    m1 = jnp.max(f, axis=1, keepdims=True)
    i1 = jnp.min(jnp.where(f == m1, iota, secl), axis=1, keepdims=True)
    f2 = jnp.where(iota == i1, -jnp.inf, f)
    m2 = jnp.max(f2, axis=1, keepdims=True)
    i2 = jnp.min(jnp.where(f2 == m2, iota, secl), axis=1, keepdims=True)
    inds_ref[...] = jnp.concatenate([i1, i2], axis=1)
    vals_ref[...] = jnp.concatenate([m1, m2], axis=1)


def _cov_feature(oh0, oh1, cov3, wcov):
    cg0 = jnp.sum(oh0[:, :, None] * cov3, axis=1)
    cg1 = jnp.sum(oh1[:, :, None] * cov3, axis=1)
    cg = jnp.concatenate([cg0, cg1], axis=1)
    return cg[:, :, None] * wcov[None, :, :]


def _attn_body(sinds_ref,
               inds_ref, vals_ref, dec_h_ref, wd_ref, bdec_ref, wv_ref,
               wcov_ref, ef_hbm, eo_hbm, cov_ref, mask_ref,
               ctx_ref, attn_ref, covout_ref,
               efg_ref, eog_ref, sem):
    bsz, secl, wordl = cov_ref.shape

    copies = []
    for b in range(bsz):
        for f in range(F):
            sec = sinds_ref[b, f]
            copies.append(pltpu.make_async_copy(
                ef_hbm.at[b, sec], efg_ref.at[b, pl.ds(f * wordl, wordl), :],
                sem))
            copies.append(pltpu.make_async_copy(
                eo_hbm.at[b, sec], eog_ref.at[b, pl.ds(f * wordl, wordl), :],
                sem))
    for c in copies:
        c.start()

    dec = lax.dot_general(
        dec_h_ref[...], wd_ref[...], (((1,), (1,)), ((), ())),
        preferred_element_type=jnp.float32)
    dec = dec + bdec_ref[...]

    inds = inds_ref[...]
    vals = vals_ref[...]
    iota_s = lax.broadcasted_iota(jnp.int32, (bsz, secl), 1)
    oh0 = (iota_s == inds[:, 0:1]).astype(jnp.float32)
    oh1 = (iota_s == inds[:, 1:2]).astype(jnp.float32)

    mask3 = mask_ref[...]
    mg0 = jnp.sum(oh0[:, :, None] * mask3, axis=1)
    mg1 = jnp.sum(oh1[:, :, None] * mask3, axis=1)
    mask_row = jnp.concatenate([mg0, mg1], axis=1)

    foc_row = jnp.concatenate(
        [jnp.broadcast_to(vals[:, 0:1], (bsz, wordl)),
         jnp.broadcast_to(vals[:, 1:2], (bsz, wordl))], axis=1)

    cov_term = _cov_feature(oh0, oh1, cov_ref[...], wcov_ref[...])

    for c in copies:
        c.wait()

    x = efg_ref[...] + dec[:, None, :] + cov_term
    t = jnp.tanh(x)
    s = jnp.sum(t * wv_ref[...][None, :, :], axis=2)

    e = jnp.exp(s - jnp.max(s, axis=1, keepdims=True))
    af = e * mask_row * foc_row
    w = af / jnp.sum(af, axis=1, keepdims=True)

    ctx_ref[...] = jnp.sum(w[:, :, None] * eog_ref[...], axis=1)

    attn = (oh0[:, :, None] * w[:, None, :wordl]
            + oh1[:, :, None] * w[:, None, wordl:])
    attn_ref[...] = attn
    covout_ref[...] = cov_ref[...] + attn


def kernel(dec_hidden, enc_output, enc_feature, enc_mask, sec_attn, coverage,
           focus, W_dec, b_dec, w_v, w_cov):
    batch, src_len, dim = enc_output.shape
    secl = focus.shape[1]
    wordl = src_len // secl

    inds, vals = pl.pallas_call(
        _topk2_body,
        out_shape=(jax.ShapeDtypeStruct((batch, F), jnp.int32),
                   jax.ShapeDtypeStruct((batch, F), jnp.float32)),
    )(focus)

    ef = enc_feature.reshape(batch, secl, wordl, dim)
    eo = enc_output.reshape(batch, secl, wordl, dim)
    cov3 = coverage.reshape(batch, secl, wordl)
    mask3 = enc_mask.reshape(batch, secl, wordl)

    grid_spec = pltpu.PrefetchScalarGridSpec(
        num_scalar_prefetch=1,
        grid=(1,),
        in_specs=[
            pl.BlockSpec(memory_space=pltpu.VMEM),
            pl.BlockSpec(memory_space=pltpu.VMEM),
            pl.BlockSpec(memory_space=pltpu.VMEM),
            pl.BlockSpec(memory_space=pltpu.VMEM),
            pl.BlockSpec(memory_space=pltpu.VMEM),
            pl.BlockSpec(memory_space=pltpu.VMEM),
            pl.BlockSpec(memory_space=pltpu.VMEM),
            pl.BlockSpec(memory_space=pltpu.HBM),
            pl.BlockSpec(memory_space=pltpu.HBM),
            pl.BlockSpec(memory_space=pltpu.VMEM),
            pl.BlockSpec(memory_space=pltpu.VMEM),
        ],
        out_specs=[
            pl.BlockSpec(memory_space=pltpu.VMEM),
            pl.BlockSpec(memory_space=pltpu.VMEM),
            pl.BlockSpec(memory_space=pltpu.VMEM),
        ],
        scratch_shapes=[
            pltpu.VMEM((batch, F * wordl, dim), jnp.float32),
            pltpu.VMEM((batch, F * wordl, dim), jnp.float32),
            pltpu.SemaphoreType.DMA,
        ],
    )

    context, attn3, covout3 = pl.pallas_call(
        _attn_body,
        grid_spec=grid_spec,
        out_shape=(jax.ShapeDtypeStruct((batch, dim), jnp.float32),
                   jax.ShapeDtypeStruct((batch, secl, wordl), jnp.float32),
                   jax.ShapeDtypeStruct((batch, secl, wordl), jnp.float32)),
    )(inds, inds, vals, dec_hidden, W_dec, b_dec.reshape(1, dim),
      w_v.reshape(1, dim), w_cov.reshape(1, dim), ef, eo, cov3, mask3)

    return (context, attn3.reshape(batch, src_len),
            covout3.reshape(batch, src_len))


# speedup vs baseline: 5.8974x; 1.0208x over previous
import jax
import jax.numpy as jnp
from jax import lax
from jax.experimental import pallas as pl
from jax.experimental.pallas import tpu as pltpu

F = 2
GROUPS = 4


def _top2(f):
    rows, cols = f.shape
    iota = lax.broadcasted_iota(jnp.int32, (rows, cols), 1)
    m1 = jnp.max(f, axis=1, keepdims=True)
    i1 = jnp.min(jnp.where(f == m1, iota, cols), axis=1, keepdims=True)
    f2 = jnp.where(iota == i1, -jnp.inf, f)
    m2 = jnp.max(f2, axis=1, keepdims=True)
    i2 = jnp.min(jnp.where(f2 == m2, iota, cols), axis=1, keepdims=True)
    return i1, i2, m1, m2, iota


def _topk2_body(focus_ref, inds_ref):
    i1, i2, _, _, _ = _top2(focus_ref[...])
    inds_ref[...] = jnp.concatenate([i1, i2], axis=1)


def _attn_body(sinds_ref,
               focus_ref, dec_h_ref, wd_ref, bdec_ref, wv_ref,
               wcov_ref, ef_hbm, eo_hbm, cov_ref, mask_ref,
               ctx_ref, attn_ref, covout_ref,
               efg_ref, eog_ref, sems):
    bsz, secl, wordl = cov_ref.shape
    gb = bsz // GROUPS

    copies = [[] for _ in range(GROUPS)]
    for b in range(bsz):
        g = b // gb
        for f in range(F):
            sec = sinds_ref[b, f]
            copies[g].append(pltpu.make_async_copy(
                ef_hbm.at[b, sec], efg_ref.at[b, pl.ds(f * wordl, wordl), :],
                sems.at[g]))
            copies[g].append(pltpu.make_async_copy(
                eo_hbm.at[b, sec], eog_ref.at[b, pl.ds(f * wordl, wordl), :],
                sems.at[g]))
    for grp in copies:
        for c in grp:
            c.start()

    i1, i2, m1, m2, iota_s = _top2(focus_ref[...])
    oh0 = (iota_s == i1).astype(jnp.float32)
    oh1 = (iota_s == i2).astype(jnp.float32)

    dec = lax.dot_general(
        dec_h_ref[...], wd_ref[...], (((1,), (1,)), ((), ())),
        preferred_element_type=jnp.float32)
    dec = dec + bdec_ref[...]

    mask3 = mask_ref[...]
    cov3 = cov_ref[...]
    mask_row = jnp.concatenate(
        [jnp.sum(oh0[:, :, None] * mask3, axis=1),
         jnp.sum(oh1[:, :, None] * mask3, axis=1)], axis=1)
    cov_row = jnp.concatenate(
        [jnp.sum(oh0[:, :, None] * cov3, axis=1),
         jnp.sum(oh1[:, :, None] * cov3, axis=1)], axis=1)
    foc_row = jnp.concatenate(
        [jnp.broadcast_to(m1, (bsz, wordl)),
         jnp.broadcast_to(m2, (bsz, wordl))], axis=1)

    wv = wv_ref[...]
    wcov = wcov_ref[...]

    for g in range(GROUPS):
        for c in copies[g]:
            c.wait()
        sl = pl.ds(g * gb, gb)
        x = (efg_ref[sl] + dec[g * gb:(g + 1) * gb, None, :]
             + cov_row[g * gb:(g + 1) * gb, :, None] * wcov[None, :, :])
        t = jnp.tanh(x)
        s = jnp.sum(t * wv[None, :, :], axis=2)

        e = jnp.exp(s - jnp.max(s, axis=1, keepdims=True))
        af = e * mask_row[g * gb:(g + 1) * gb] * foc_row[g * gb:(g + 1) * gb]
        w = af / jnp.sum(af, axis=1, keepdims=True)

        ctx_ref[sl] = jnp.sum(w[:, :, None] * eog_ref[sl], axis=1)

        attn = (oh0[g * gb:(g + 1) * gb, :, None] * w[:, None, :wordl]
                + oh1[g * gb:(g + 1) * gb, :, None] * w[:, None, wordl:])
        attn_ref[sl] = attn
        covout_ref[sl] = cov3[g * gb:(g + 1) * gb] + attn


def kernel(dec_hidden, enc_output, enc_feature, enc_mask, sec_attn, coverage,
           focus, W_dec, b_dec, w_v, w_cov):
    batch, src_len, dim = enc_output.shape
    secl = focus.shape[1]
    wordl = src_len // secl

    inds = pl.pallas_call(
        _topk2_body,
        out_shape=jax.ShapeDtypeStruct((batch, F), jnp.int32),
    )(focus)

    ef = enc_feature.reshape(batch, secl, wordl, dim)
    eo = enc_output.reshape(batch, secl, wordl, dim)
    cov3 = coverage.reshape(batch, secl, wordl)
    mask3 = enc_mask.reshape(batch, secl, wordl)

    grid_spec = pltpu.PrefetchScalarGridSpec(
        num_scalar_prefetch=1,
        grid=(1,),
        in_specs=[
            pl.BlockSpec(memory_space=pltpu.VMEM),
            pl.BlockSpec(memory_space=pltpu.VMEM),
            pl.BlockSpec(memory_space=pltpu.VMEM),
            pl.BlockSpec(memory_space=pltpu.VMEM),
            pl.BlockSpec(memory_space=pltpu.VMEM),
            pl.BlockSpec(memory_space=pltpu.VMEM),
            pl.BlockSpec(memory_space=pltpu.HBM),
            pl.BlockSpec(memory_space=pltpu.HBM),
            pl.BlockSpec(memory_space=pltpu.VMEM),
            pl.BlockSpec(memory_space=pltpu.VMEM),
        ],
        out_specs=[
            pl.BlockSpec(memory_space=pltpu.VMEM),
            pl.BlockSpec(memory_space=pltpu.VMEM),
            pl.BlockSpec(memory_space=pltpu.VMEM),
        ],
        scratch_shapes=[
            pltpu.VMEM((batch, F * wordl, dim), jnp.float32),
            pltpu.VMEM((batch, F * wordl, dim), jnp.float32),
            pltpu.SemaphoreType.DMA((GROUPS,)),
        ],
    )

    context, attn3, covout3 = pl.pallas_call(
        _attn_body,
        grid_spec=grid_spec,
        out_shape=(jax.ShapeDtypeStruct((batch, dim), jnp.float32),
                   jax.ShapeDtypeStruct((batch, secl, wordl), jnp.float32),
                   jax.ShapeDtypeStruct((batch, secl, wordl), jnp.float32)),
    )(inds, focus, dec_hidden, W_dec, b_dec.reshape(1, dim),
      w_v.reshape(1, dim), w_cov.reshape(1, dim), ef, eo, cov3, mask3)

    return (context, attn3.reshape(batch, src_len),
            covout3.reshape(batch, src_len))


# speedup vs baseline: 6.3678x; 1.0798x over previous
import jax
import jax.numpy as jnp
from jax import lax
from jax.experimental import pallas as pl
from jax.experimental.pallas import tpu as pltpu

F = 2
GROUPS = 4


def _top2(f):
    rows, cols = f.shape
    iota = lax.broadcasted_iota(jnp.int32, (rows, cols), 1)
    m1 = jnp.max(f, axis=1, keepdims=True)
    i1 = jnp.min(jnp.where(f == m1, iota, cols), axis=1, keepdims=True)
    f2 = jnp.where(iota == i1, -jnp.inf, f)
    m2 = jnp.max(f2, axis=1, keepdims=True)
    i2 = jnp.min(jnp.where(f2 == m2, iota, cols), axis=1, keepdims=True)
    return i1, i2, m1, m2, iota


def _attn_body(focus_ref, dec_h_ref, wd_ref, bdec_ref, wv_ref,
               wcov_ref, ef_hbm, eo_hbm, cov_ref, mask_ref,
               ctx_ref, attn_ref, covout_ref,
               efg_ref, eog_ref, iv_ref, is_ref, sems, isem):
    bsz, secl, wordl = cov_ref.shape
    gb = bsz // GROUPS

    i1, i2, m1, m2, iota_s = _top2(focus_ref[...])
    iv_ref[...] = jnp.concatenate([i1, i2], axis=1)
    idx_copy = pltpu.make_async_copy(iv_ref, is_ref, isem)
    idx_copy.start()

    oh0 = (iota_s == i1).astype(jnp.float32)
    oh1 = (iota_s == i2).astype(jnp.float32)

    idx_copy.wait()

    copies = [[] for _ in range(GROUPS)]
    for b in range(bsz):
        g = b // gb
        for f in range(F):
            sec = is_ref[b, f]
            copies[g].append(pltpu.make_async_copy(
                ef_hbm.at[b, sec], efg_ref.at[b, pl.ds(f * wordl, wordl), :],
                sems.at[g]))
            copies[g].append(pltpu.make_async_copy(
                eo_hbm.at[b, sec], eog_ref.at[b, pl.ds(f * wordl, wordl), :],
                sems.at[g]))
    for grp in copies:
        for c in grp:
            c.start()

    dec = lax.dot_general(
        dec_h_ref[...], wd_ref[...], (((1,), (1,)), ((), ())),
        preferred_element_type=jnp.float32)
    dec = dec + bdec_ref[...]

    mask3 = mask_ref[...]
    cov3 = cov_ref[...]
    mask_row = jnp.concatenate(
        [jnp.sum(oh0[:, :, None] * mask3, axis=1),
         jnp.sum(oh1[:, :, None] * mask3, axis=1)], axis=1)
    cov_row = jnp.concatenate(
        [jnp.sum(oh0[:, :, None] * cov3, axis=1),
         jnp.sum(oh1[:, :, None] * cov3, axis=1)], axis=1)
    foc_row = jnp.concatenate(
        [jnp.broadcast_to(m1, (bsz, wordl)),
         jnp.broadcast_to(m2, (bsz, wordl))], axis=1)

    wv = wv_ref[...]
    wcov = wcov_ref[...]

    for g in range(GROUPS):
        for c in copies[g]:
            c.wait()
        sl = pl.ds(g * gb, gb)
        x = (efg_ref[sl] + dec[g * gb:(g + 1) * gb, None, :]
             + cov_row[g * gb:(g + 1) * gb, :, None] * wcov[None, :, :])
        t = jnp.tanh(x)
        s = jnp.sum(t * wv[None, :, :], axis=2)

        e = jnp.exp(s - jnp.max(s, axis=1, keepdims=True))
        af = e * mask_row[g * gb:(g + 1) * gb] * foc_row[g * gb:(g + 1) * gb]
        w = af / jnp.sum(af, axis=1, keepdims=True)

        ctx_ref[sl] = jnp.sum(w[:, :, None] * eog_ref[sl], axis=1)

        attn = (oh0[g * gb:(g + 1) * gb, :, None] * w[:, None, :wordl]
                + oh1[g * gb:(g + 1) * gb, :, None] * w[:, None, wordl:])
        attn_ref[sl] = attn
        covout_ref[sl] = cov3[g * gb:(g + 1) * gb] + attn


def kernel(dec_hidden, enc_output, enc_feature, enc_mask, sec_attn, coverage,
           focus, W_dec, b_dec, w_v, w_cov):
    batch, src_len, dim = enc_output.shape
    secl = focus.shape[1]
    wordl = src_len // secl

    ef = enc_feature.reshape(batch, secl, wordl, dim)
    eo = enc_output.reshape(batch, secl, wordl, dim)
    cov3 = coverage.reshape(batch, secl, wordl)
    mask3 = enc_mask.reshape(batch, secl, wordl)

    context, attn3, covout3 = pl.pallas_call(
        _attn_body,
        in_specs=[
            pl.BlockSpec(memory_space=pltpu.VMEM),
            pl.BlockSpec(memory_space=pltpu.VMEM),
            pl.BlockSpec(memory_space=pltpu.VMEM),
            pl.BlockSpec(memory_space=pltpu.VMEM),
            pl.BlockSpec(memory_space=pltpu.VMEM),
            pl.BlockSpec(memory_space=pltpu.VMEM),
            pl.BlockSpec(memory_space=pltpu.HBM),
            pl.BlockSpec(memory_space=pltpu.HBM),
            pl.BlockSpec(memory_space=pltpu.VMEM),
            pl.BlockSpec(memory_space=pltpu.VMEM),
        ],
        out_specs=[
            pl.BlockSpec(memory_space=pltpu.VMEM),
            pl.BlockSpec(memory_space=pltpu.VMEM),
            pl.BlockSpec(memory_space=pltpu.VMEM),
        ],
        scratch_shapes=[
            pltpu.VMEM((batch, F * wordl, dim), jnp.float32),
            pltpu.VMEM((batch, F * wordl, dim), jnp.float32),
            pltpu.VMEM((batch, F), jnp.int32),
            pltpu.SMEM((batch, F), jnp.int32),
            pltpu.SemaphoreType.DMA((GROUPS,)),
            pltpu.SemaphoreType.DMA,
        ],
        out_shape=(jax.ShapeDtypeStruct((batch, dim), jnp.float32),
                   jax.ShapeDtypeStruct((batch, secl, wordl), jnp.float32),
                   jax.ShapeDtypeStruct((batch, secl, wordl), jnp.float32)),
    )(focus, dec_hidden, W_dec, b_dec.reshape(1, dim),
      w_v.reshape(1, dim), w_cov.reshape(1, dim), ef, eo, cov3, mask3)

    return (context, attn3.reshape(batch, src_len),
            covout3.reshape(batch, src_len))
